# baseline (speedup 1.0000x reference)
import math
import functools

import jax
import jax.numpy as jnp
from jax import lax
from jax.experimental import pallas as pl
from jax.experimental.pallas import tpu as pltpu
from jax.experimental.pallas import tpu_sc as plsc

VOCAB = 1000000
EMBED = 64
B, L = 4096, 200
SCALE = math.sqrt(EMBED)

NC, NS = 2, 16
NW = NC * NS

NTC = VOCAB // 128
NTC_LAST = VOCAB - NTC * 128
BLKS_EVEN = NTC // NW
BLKS_REM = NTC - BLKS_EVEN * NW
SUBL = 2
BBLK = B // NW
NSTEP = L // SUBL

_MESH = dict(core_axis_name="c", subcore_axis_name="s")


def _pe_table():
    idx = jnp.arange(0, EMBED, 2, dtype=jnp.float32)
    pos = jnp.arange(0, L, dtype=jnp.float32)[:, None]
    div_term = jnp.exp(-idx / EMBED * math.log(10000.0))
    ang = pos * div_term
    pe = jnp.zeros((L, EMBED), dtype=jnp.float32)
    pe = pe.at[:, 0::2].set(jnp.sin(ang))
    pe = pe.at[:, 1::2].set(jnp.cos(ang))
    return pe


@functools.partial(
    pl.kernel,
    out_type=jax.ShapeDtypeStruct((VOCAB // 2, 128), jnp.float32),
    mesh=plsc.VectorSubcoreMesh(**_MESH),
    scratch_types=[
        pltpu.VMEM((EMBED, 128), jnp.float32),
        pltpu.VMEM((EMBED, 128), jnp.float32),
        pltpu.VMEM((EMBED, 128), jnp.float32),
        pltpu.VMEM((EMBED, 128), jnp.float32),
        pltpu.VMEM((EMBED, EMBED), jnp.float32),
        pltpu.SemaphoreType.DMA,
        pltpu.SemaphoreType.DMA,
        pltpu.SemaphoreType.DMA,
        pltpu.SemaphoreType.DMA,
    ],
    compiler_params=pltpu.CompilerParams(needs_layout_passes=False),
)
def _sc_repack(wt_hbm, wp_hbm, in0, in1, ot0, ot1, in64, is0, is1, os0, os1):
    wid = lax.axis_index("s") * NC + lax.axis_index("c")
    iota = lax.iota(jnp.int32, 16)
    rows16 = [iota + 16 * j for j in range(4)]

    ins = (in0, in1)
    outs = (ot0, ot1)
    isems = (is0, is1)
    osems = (os0, os1)

    def fire_read(i, slot):
        c = wid + i * NW
        col0 = pl.multiple_of(c * 128, 128)
        pltpu.async_copy(wt_hbm.at[:, pl.ds(col0, 128)], ins[slot], isems[slot])

    def wait_read(slot):
        pltpu.make_async_copy(
            wt_hbm.at[:, pl.ds(0, 128)], ins[slot], isems[slot]
        ).wait()

    vvl = [iota + vb * 16 for vb in range(8)]
    vv64 = [(iota + vb * 16) * 64 for vb in range(8)]

    def transpose(in_v, out_v):
        @plsc.parallel_loop(0, 64, unroll=2)
        def e_body(e):
            for vb in range(8):
                vals = in_v[e, pl.ds(vb * 16, 16)]
                flat = vv64[vb] + ((vvl[vb] + e) & 63)
                plsc.store_scatter(out_v, [flat >> 7, flat & 127], vals)

    def fire_write(i, slot):
        c = wid + i * NW
        row0 = pl.multiple_of(c * 64, 64)
        pltpu.async_copy(outs[slot], wp_hbm.at[pl.ds(row0, 64)], osems[slot])

    def wait_write(slot):
        pltpu.make_async_copy(
            outs[slot], wp_hbm.at[pl.ds(0, 64)], osems[slot]
        ).wait()

    fire_read(0, 0)

    def g_body(g, carry):
        i0 = g * 2
        fire_read(i0 + 1, 1)
        wait_read(0)
        transpose(in0, ot0)

        @pl.when(g > 0)
        def _():
            wait_write(0)

        fire_write(i0, 0)

        @pl.when(g < BLKS_EVEN // 2 - 1)
        def _():
            fire_read(i0 + 2, 0)

        wait_read(1)
        transpose(in1, ot1)

        @pl.when(g > 0)
        def _():
            wait_write(1)

        fire_write(i0 + 1, 1)
        return carry

    lax.fori_loop(0, BLKS_EVEN // 2, g_body, 0)
    wait_write(0)
    wait_write(1)

    @pl.when(wid < BLKS_REM)
    def _extra():
        c = wid + BLKS_EVEN * NW
        col0 = pl.multiple_of(c * 128, 128)
        row0 = pl.multiple_of(c * 64, 64)
        pltpu.sync_copy(wt_hbm.at[:, pl.ds(col0, 128)], in0)
        transpose(in0, ot0)
        pltpu.sync_copy(ot0, wp_hbm.at[pl.ds(row0, 64)])

    @pl.when(wid == BLKS_REM)
    def _tail():
        pltpu.sync_copy(wt_hbm.at[:, pl.ds(NTC * 128, NTC_LAST)], in64)

        @plsc.parallel_loop(0, 64, unroll=2)
        def e_body(e):
            for vb in range(NTC_LAST // 16):
                vals = in64[e, pl.ds(vb * 16, 16)]
                flat = vv64[vb] + ((vvl[vb] + e) & 63)
                plsc.store_scatter(ot0, [flat >> 7, flat & 127], vals)
        pltpu.sync_copy(
            ot0.at[pl.ds(0, NTC_LAST // 2)],
            wp_hbm.at[pl.ds(NTC * 64, NTC_LAST // 2)],
        )


@functools.partial(
    pl.kernel,
    out_type=jax.ShapeDtypeStruct((L, 8, NW, 1024), jnp.float32),
    mesh=plsc.VectorSubcoreMesh(**_MESH),
    scratch_types=[
        pltpu.VMEM((SUBL, BBLK), jnp.int32),
        pltpu.VMEM((SUBL, BBLK), jnp.int32),
        pltpu.VMEM((SUBL * BBLK, EMBED), jnp.float32),
        pltpu.VMEM((SUBL * BBLK, EMBED), jnp.float32),
        pltpu.VMEM((SUBL, 8, 1, 1024), jnp.float32),
        pltpu.VMEM((SUBL, 8, 1, 1024), jnp.float32),
        pltpu.VMEM((L, EMBED), jnp.float32),
        pltpu.SemaphoreType.DMA,
        pltpu.SemaphoreType.DMA,
        pltpu.SemaphoreType.DMA,
        pltpu.SemaphoreType.DMA,
    ],
    compiler_params=pltpu.CompilerParams(
        use_tc_tiling_on_sc=False, needs_layout_passes=False
    ),
)
def _sc_embed(
    xt_hbm, w_hbm, pe_hbm, out_hbm,
    idx0, idx1, rows0, rows1, out0, out1, pe_v,
    gs0, gs1, os0, os1,
):
    wid = lax.axis_index("s") * NC + lax.axis_index("c")
    b0 = wid * BBLK
    iota = lax.iota(jnp.int32, 16)

    idxs = (idx0, idx1)
    rows = (rows0, rows1)
    outs = (out0, out1)
    gsems = (gs0, gs1)
    osems = (os0, os1)

    pltpu.sync_copy(pe_hbm, pe_v)

    def fire_gathers(t, slot):
        l0 = t * SUBL
        pltpu.sync_copy(xt_hbm.at[pl.ds(l0, SUBL), pl.ds(b0, BBLK)], idxs[slot])
        for s in range(SUBL):
            pltpu.async_copy(
                w_hbm.at[idxs[slot].at[s]],
                rows[slot].at[pl.ds(s * BBLK, BBLK)],
                gsems[slot],
            )

    def wait_gathers(slot):
        for s in range(SUBL):
            pltpu.make_async_copy(
                w_hbm.at[pl.ds(0, BBLK)],
                rows[slot].at[pl.ds(s * BBLK, BBLK)],
                gsems[slot],
            ).wait()

    def compute(t, slot):
        l0 = t * SUBL
        rows_v = rows[slot]
        out_v = outs[slot]
        rowls = [jnp.full((16,), 0, jnp.int32) + (l0 + s) for s in range(SUBL)]
        ids = [
            idxs[slot][s, pl.ds(bg * 16, 16)]
            for s in range(SUBL)
            for bg in range(BBLK // 16)
        ]

        @plsc.parallel_loop(0, 8, unroll=4)
        def te_body(te):
            for r in range(8):
                colev = jnp.full((16,), 0, jnp.int32) + (te * 8 + r)
                for s in range(SUBL):
                    pe_s = plsc.load_gather(pe_v, [rowls[s], colev])
                    for bg in range(BBLK // 16):
                        r16 = iota + (s * BBLK + bg * 16)
                        sw = (colev + ids[s * (BBLK // 16) + bg]) & 63
                        vals = plsc.load_gather(rows_v, [r16, sw])
                        out_v[s, te, 0, pl.ds(r * 128 + bg * 16, 16)] = (
                            vals * SCALE + pe_s
                        )

    def fire_store(t, slot):
        l0 = t * SUBL
        pltpu.async_copy(
            outs[slot],
            out_hbm.at[pl.ds(l0, SUBL), pl.ds(0, 8), pl.ds(wid, 1), pl.ds(0, 1024)],
            osems[slot],
        )

    def wait_store(slot):
        pltpu.make_async_copy(
            outs[slot],
            out_hbm.at[pl.ds(0, SUBL), pl.ds(0, 8), pl.ds(wid, 1), pl.ds(0, 1024)],
            osems[slot],
        ).wait()

    fire_gathers(0, 0)

    def g_body(g, carry):
        t0 = g * 2
        fire_gathers(t0 + 1, 1)
        wait_gathers(0)
        compute(t0, 0)

        @pl.when(g > 0)
        def _():
            wait_store(0)

        fire_store(t0, 0)

        @pl.when(g < NSTEP // 2 - 1)
        def _():
            fire_gathers(t0 + 2, 0)

        wait_gathers(1)
        compute(t0 + 1, 1)

        @pl.when(g > 0)
        def _():
            wait_store(1)

        fire_store(t0 + 1, 1)
        return carry

    lax.fori_loop(0, NSTEP // 2, g_body, 0)
    wait_store(0)
    wait_store(1)


def kernel(x, W):
    pe = _pe_table()
    wt = jnp.swapaxes(W, 0, 1)
    xt = jnp.swapaxes(x, 0, 1)
    wp = _sc_repack(wt)
    w64 = wp.reshape(VOCAB, EMBED)
    o4 = _sc_embed(xt, w64, pe)
    out = (
        o4.reshape(L, 8, NW, 8, 128)
        .transpose(2, 4, 0, 1, 3)
        .reshape(B, L, EMBED)
    )
    return out


# speedup vs baseline: 1.6379x; 1.6379x over previous
import math
import functools

import jax
import jax.numpy as jnp
from jax import lax
from jax.experimental import pallas as pl
from jax.experimental.pallas import tpu as pltpu
from jax.experimental.pallas import tpu_sc as plsc

VOCAB = 1000000
EMBED = 64
B, L = 4096, 200
SCALE = math.sqrt(EMBED)

NC, NS = 2, 16
NW = NC * NS

NTC = VOCAB // 128
NTC_LAST = VOCAB - NTC * 128
BLKS_EVEN = NTC // NW
BLKS_REM = NTC - BLKS_EVEN * NW
SUBL = 2
BBLK = B // NW
NSTEP = L // SUBL

_MESH = dict(core_axis_name="c", subcore_axis_name="s")


def _pe_table():
    idx = jnp.arange(0, EMBED, 2, dtype=jnp.float32)
    pos = jnp.arange(0, L, dtype=jnp.float32)[:, None]
    div_term = jnp.exp(-idx / EMBED * math.log(10000.0))
    ang = pos * div_term
    pe = jnp.zeros((L, EMBED), dtype=jnp.float32)
    pe = pe.at[:, 0::2].set(jnp.sin(ang))
    pe = pe.at[:, 1::2].set(jnp.cos(ang))
    return pe


@functools.partial(
    pl.kernel,
    out_type=jax.ShapeDtypeStruct((VOCAB // 2, 128), jnp.float32),
    mesh=plsc.VectorSubcoreMesh(**_MESH),
    scratch_types=[
        pltpu.VMEM((EMBED, 128), jnp.float32),
        pltpu.VMEM((EMBED, 128), jnp.float32),
        pltpu.VMEM((EMBED, 128), jnp.float32),
        pltpu.VMEM((EMBED, 128), jnp.float32),
        pltpu.VMEM((EMBED, EMBED), jnp.float32),
        pltpu.SemaphoreType.DMA,
        pltpu.SemaphoreType.DMA,
        pltpu.SemaphoreType.DMA,
        pltpu.SemaphoreType.DMA,
    ],
    compiler_params=pltpu.CompilerParams(needs_layout_passes=False),
)
def _sc_repack(wt_hbm, wp_hbm, in0, in1, ot0, ot1, in64, is0, is1, os0, os1):
    wid = lax.axis_index("s") * NC + lax.axis_index("c")
    iota = lax.iota(jnp.int32, 16)
    rows16 = [iota + 16 * j for j in range(4)]

    ins = (in0, in1)
    outs = (ot0, ot1)
    isems = (is0, is1)
    osems = (os0, os1)

    def fire_read(i, slot):
        c = wid + i * NW
        col0 = pl.multiple_of(c * 128, 128)
        pltpu.async_copy(wt_hbm.at[:, pl.ds(col0, 128)], ins[slot], isems[slot])

    def wait_read(slot):
        pltpu.make_async_copy(
            wt_hbm.at[:, pl.ds(0, 128)], ins[slot], isems[slot]
        ).wait()

    vvl = [iota + vb * 16 for vb in range(8)]
    vv64 = [(iota + vb * 16) * 64 for vb in range(8)]

    def transpose(in_v, out_v):
        @plsc.parallel_loop(0, 64, unroll=2)
        def e_body(e):
            for vb in range(8):
                vals = in_v[e, pl.ds(vb * 16, 16)]
                flat = vv64[vb] + ((vvl[vb] + e) & 63)
                plsc.store_scatter(out_v, [flat >> 7, flat & 127], vals)

    def fire_write(i, slot):
        c = wid + i * NW
        row0 = pl.multiple_of(c * 64, 64)
        pltpu.async_copy(outs[slot], wp_hbm.at[pl.ds(row0, 64)], osems[slot])

    def wait_write(slot):
        pltpu.make_async_copy(
            outs[slot], wp_hbm.at[pl.ds(0, 64)], osems[slot]
        ).wait()

    fire_read(0, 0)

    def g_body(g, carry):
        i0 = g * 2
        fire_read(i0 + 1, 1)
        wait_read(0)
        transpose(in0, ot0)

        @pl.when(g > 0)
        def _():
            wait_write(0)

        fire_write(i0, 0)

        @pl.when(g < BLKS_EVEN // 2 - 1)
        def _():
            fire_read(i0 + 2, 0)

        wait_read(1)
        transpose(in1, ot1)

        @pl.when(g > 0)
        def _():
            wait_write(1)

        fire_write(i0 + 1, 1)
        return carry

    lax.fori_loop(0, BLKS_EVEN // 2, g_body, 0)
    wait_write(0)
    wait_write(1)

    @pl.when(wid < BLKS_REM)
    def _extra():
        c = wid + BLKS_EVEN * NW
        col0 = pl.multiple_of(c * 128, 128)
        row0 = pl.multiple_of(c * 64, 64)
        pltpu.sync_copy(wt_hbm.at[:, pl.ds(col0, 128)], in0)
        transpose(in0, ot0)
        pltpu.sync_copy(ot0, wp_hbm.at[pl.ds(row0, 64)])

    @pl.when(wid == BLKS_REM)
    def _tail():
        pltpu.sync_copy(wt_hbm.at[:, pl.ds(NTC * 128, NTC_LAST)], in64)

        @plsc.parallel_loop(0, 64, unroll=2)
        def e_body(e):
            for vb in range(NTC_LAST // 16):
                vals = in64[e, pl.ds(vb * 16, 16)]
                flat = vv64[vb] + ((vvl[vb] + e) & 63)
                plsc.store_scatter(ot0, [flat >> 7, flat & 127], vals)
        pltpu.sync_copy(
            ot0.at[pl.ds(0, NTC_LAST // 2)],
            wp_hbm.at[pl.ds(NTC * 64, NTC_LAST // 2)],
        )


@functools.partial(
    pl.kernel,
    out_type=jax.ShapeDtypeStruct((L, 8, NW, 1024), jnp.float32),
    mesh=plsc.VectorSubcoreMesh(**_MESH),
    scratch_types=[
        pltpu.VMEM((SUBL, BBLK), jnp.int32),
        pltpu.VMEM((SUBL, BBLK), jnp.int32),
        pltpu.VMEM((SUBL * BBLK, EMBED), jnp.float32),
        pltpu.VMEM((SUBL * BBLK, EMBED), jnp.float32),
        pltpu.VMEM((SUBL, 8, 1, 1024), jnp.float32),
        pltpu.VMEM((SUBL, 8, 1, 1024), jnp.float32),
        pltpu.VMEM((L, EMBED), jnp.float32),
        pltpu.SemaphoreType.DMA,
        pltpu.SemaphoreType.DMA,
        pltpu.SemaphoreType.DMA,
        pltpu.SemaphoreType.DMA,
    ],
    compiler_params=pltpu.CompilerParams(
        use_tc_tiling_on_sc=False, needs_layout_passes=False
    ),
)
def _sc_embed(
    xt_hbm, w_hbm, pe_hbm, out_hbm,
    idx0, idx1, rows0, rows1, out0, out1, pe_v,
    gs0, gs1, os0, os1,
):
    wid = lax.axis_index("s") * NC + lax.axis_index("c")
    b0 = wid * BBLK
    iota = lax.iota(jnp.int32, 16)

    idxs = (idx0, idx1)
    rows = (rows0, rows1)
    outs = (out0, out1)
    gsems = (gs0, gs1)
    osems = (os0, os1)

    pltpu.sync_copy(pe_hbm, pe_v)

    def fire_gathers(t, slot):
        l0 = t * SUBL
        pltpu.sync_copy(xt_hbm.at[pl.ds(l0, SUBL), pl.ds(b0, BBLK)], idxs[slot])
        for s in range(SUBL):
            pltpu.async_copy(
                w_hbm.at[idxs[slot].at[s]],
                rows[slot].at[pl.ds(s * BBLK, BBLK)],
                gsems[slot],
            )

    def wait_gathers(slot):
        for s in range(SUBL):
            pltpu.make_async_copy(
                w_hbm.at[pl.ds(0, BBLK)],
                rows[slot].at[pl.ds(s * BBLK, BBLK)],
                gsems[slot],
            ).wait()

    def compute(t, slot):
        l0 = t * SUBL
        rows_v = rows[slot]
        out_v = outs[slot]
        rowls = [jnp.full((16,), 0, jnp.int32) + (l0 + s) for s in range(SUBL)]
        ids = [
            idxs[slot][s, pl.ds(bg * 16, 16)]
            for s in range(SUBL)
            for bg in range(BBLK // 16)
        ]

        @plsc.parallel_loop(0, 8, unroll=2)
        def te_body(te):
            for r in range(8):
                colev = jnp.full((16,), 0, jnp.int32) + (te * 8 + r)
                for s in range(SUBL):
                    pe_s = plsc.load_gather(pe_v, [rowls[s], colev])
                    for bg in range(BBLK // 16):
                        r16 = iota + (s * BBLK + bg * 16)
                        sw = (colev + ids[s * (BBLK // 16) + bg]) & 63
                        vals = plsc.load_gather(rows_v, [r16, sw])
                        out_v[s, te, 0, pl.ds(r * 128 + bg * 16, 16)] = (
                            vals * SCALE + pe_s
                        )

    def fire_store(t, slot):
        l0 = t * SUBL
        pltpu.async_copy(
            outs[slot],
            out_hbm.at[pl.ds(l0, SUBL), pl.ds(0, 8), pl.ds(wid, 1), pl.ds(0, 1024)],
            osems[slot],
        )

    def wait_store(slot):
        pltpu.make_async_copy(
            outs[slot],
            out_hbm.at[pl.ds(0, SUBL), pl.ds(0, 8), pl.ds(wid, 1), pl.ds(0, 1024)],
            osems[slot],
        ).wait()

    fire_gathers(0, 0)

    def g_body(g, carry):
        t0 = g * 2
        fire_gathers(t0 + 1, 1)
        wait_gathers(0)
        compute(t0, 0)

        @pl.when(g > 0)
        def _():
            wait_store(0)

        fire_store(t0, 0)

        @pl.when(g < NSTEP // 2 - 1)
        def _():
            fire_gathers(t0 + 2, 0)

        wait_gathers(1)
        compute(t0 + 1, 1)

        @pl.when(g > 0)
        def _():
            wait_store(1)

        fire_store(t0 + 1, 1)
        return carry

    lax.fori_loop(0, NSTEP // 2, g_body, 0)
    wait_store(0)
    wait_store(1)


def kernel(x, W):
    pe = _pe_table()
    wt = jnp.swapaxes(W, 0, 1)
    xt = jnp.swapaxes(x, 0, 1)
    wp = _sc_repack(wt)
    w64 = wp.reshape(VOCAB, EMBED)
    o4 = _sc_embed(xt, w64, pe)
    out = (
        o4.reshape(L, 8, NW, 8, 128)
        .transpose(2, 4, 0, 1, 3)
        .reshape(B, L, EMBED)
    )
    return out
